# initial kernel scaffold (unmeasured)
import jax
import jax.numpy as jnp
from jax import lax
from jax.experimental import pallas as pl
from jax.experimental.pallas import tpu as pltpu

B = 32
H = 16
D = 128
BS = 32
NB = 256
NPG = 512
NPL = 256
CP = 32
T = CP * BS
NC = NPL // CP
SCALE = D ** -0.5
NEG = -1e30


def kernel(Q, K, V, bt, lens):
    q3 = Q.reshape(B, H, D)
    lens2 = lens.reshape(B, 1)

    def body(q_ref, k_ref, v_ref, bt_ref, lens_ref, out_ref,
             m_ref, l_ref, acc_ref, rm_ref, rl_ref, racc_ref,
             send_sems, recv_sems):
        c = pl.program_id(0)
        my_x = lax.axis_index("x")
        my_y = lax.axis_index("y")
        my_z = lax.axis_index("z")

        @pl.when(c == 0)
        def _():
            m_ref[...] = jnp.full((H, B), NEG, jnp.float32)
            l_ref[...] = jnp.zeros((H, B), jnp.float32)
            acc_ref[...] = jnp.zeros((H, B, D), jnp.float32)

        valid = lax.broadcasted_iota(jnp.int32, (B, NB), 1) < lens_ref[...]
        btv = jnp.where(valid, bt_ref[...], -1)
        base = my_x * NPL + c * CP
        pids = base + lax.broadcasted_iota(jnp.int32, (1, 1, CP), 2)
        w = jnp.sum((btv[:, :, None] == pids).astype(jnp.float32), axis=1)
        wt = jnp.broadcast_to(w[:, :, None], (B, CP, BS)).reshape(B, T)

        kc = k_ref[...].reshape(T, H, D)
        vc = v_ref[...].reshape(T, H, D)
        s = lax.dot_general(
            q_ref[...], kc, (((2,), (2,)), ((1,), (1,))),
            preferred_element_type=jnp.float32,
        ) * SCALE
        sm = jnp.where((wt > 0)[None], s, NEG)
        m_old = m_ref[...]
        m_new = jnp.maximum(m_old, jnp.max(sm, axis=2))
        alpha = jnp.exp(m_old - m_new)
        p = jnp.exp(sm - m_new[:, :, None]) * wt[None]
        l_ref[...] = l_ref[...] * alpha + jnp.sum(p, axis=2)
        acc_ref[...] = acc_ref[...] * alpha[:, :, None] + lax.dot_general(
            p, vc, (((2,), (0,)), ((0,), (1,))),
            preferred_element_type=jnp.float32,
        )
        m_ref[...] = m_new

        @pl.when(c == NC - 1)
        def _():
            partner = (1 - my_x, my_y, my_z)
            barrier = pltpu.get_barrier_semaphore()
            pl.semaphore_signal(barrier, inc=1, device_id=partner,
                                device_id_type=pl.DeviceIdType.MESH)
            pl.semaphore_wait(barrier, 1)

            copies = []
            for k_, (src, dst) in enumerate(
                    ((m_ref, rm_ref), (l_ref, rl_ref), (acc_ref, racc_ref))):
                rdma = pltpu.make_async_remote_copy(
                    src_ref=src, dst_ref=dst,
                    send_sem=send_sems.at[k_], recv_sem=recv_sems.at[k_],
                    device_id=partner, device_id_type=pl.DeviceIdType.MESH)
                rdma.start()
                copies.append(rdma)
            for rdma in copies:
                rdma.wait()

            m_l, l_l, a_l = m_ref[...], l_ref[...], acc_ref[...]
            m_r, l_r, a_r = rm_ref[...], rl_ref[...], racc_ref[...]
            m = jnp.maximum(m_l, m_r)
            c_l = jnp.exp(m_l - m)
            c_r = jnp.exp(m_r - m)
            l = l_l * c_l + l_r * c_r
            o = (a_l * c_l[:, :, None] + a_r * c_r[:, :, None]) / l[:, :, None]
            out_ref[...] = jnp.transpose(o, (1, 0, 2)).reshape(B, 1, H, D)

    return pl.pallas_call(
        body,
        grid=(NC,),
        in_specs=[
            pl.BlockSpec((B, H, D), lambda c: (0, 0, 0)),
            pl.BlockSpec((CP, BS, H, D), lambda c: (c, 0, 0, 0)),
            pl.BlockSpec((CP, BS, H, D), lambda c: (c, 0, 0, 0)),
            pl.BlockSpec((B, NB), lambda c: (0, 0)),
            pl.BlockSpec((B, 1), lambda c: (0, 0)),
        ],
        out_specs=pl.BlockSpec((B, 1, H, D), lambda c: (0, 0, 0, 0)),
        out_shape=jax.ShapeDtypeStruct((B, 1, H, D), jnp.float32),
        scratch_shapes=[
            pltpu.VMEM((H, B), jnp.float32),
            pltpu.VMEM((H, B), jnp.float32),
            pltpu.VMEM((H, B, D), jnp.float32),
            pltpu.VMEM((H, B), jnp.float32),
            pltpu.VMEM((H, B), jnp.float32),
            pltpu.VMEM((H, B, D), jnp.float32),
            pltpu.SemaphoreType.DMA((3,)),
            pltpu.SemaphoreType.DMA((3,)),
        ],
        compiler_params=pltpu.CompilerParams(collective_id=0),
    )(q3, K, V, bt, lens2)


# baseline (device time: 151335 ns/iter reference)
import jax
import jax.numpy as jnp
from jax import lax
from jax.experimental import pallas as pl
from jax.experimental.pallas import tpu as pltpu

B = 32
H = 16
D = 128
BS = 32
NB = 256
NPL = 256
CP = 16
T = CP * BS
NC = NPL // CP
SCALE = D ** -0.5
NEG = -1e30


def kernel(Q, K, V, bt, lens):
    q3 = Q.reshape(B, H, D)
    lens2 = lens.reshape(B, 1)

    def body(q_ref, k_ref, v_ref, bt_ref, lens_ref, out_ref,
             m_ref, l_ref, acc_ref, rm_ref, rl_ref, racc_ref,
             send_sems, recv_sems):
        c = pl.program_id(0)
        my_x = lax.axis_index("x")
        my_y = lax.axis_index("y")
        my_z = lax.axis_index("z")

        @pl.when(c == 0)
        def _():
            m_ref[...] = jnp.full((H, B, 1), NEG, jnp.float32)
            l_ref[...] = jnp.zeros((H, B, 1), jnp.float32)
            acc_ref[...] = jnp.zeros((H, B, D), jnp.float32)

        valid = lax.broadcasted_iota(jnp.int32, (B, NB), 1) < lens_ref[...]
        btv = jnp.where(valid, bt_ref[...], -1)
        base = my_x * NPL + c * CP
        pids = base + lax.broadcasted_iota(jnp.int32, (1, 1, CP), 2)
        w = jnp.sum((btv[:, :, None] == pids).astype(jnp.float32), axis=1)
        wt = jnp.broadcast_to(w[:, :, None], (B, CP, BS)).reshape(B, T)
        live = wt > 0

        for h in range(H):
            kc = k_ref[:, :, h, :].reshape(T, D)
            vc = v_ref[:, :, h, :].reshape(T, D)
            qh = q_ref[:, h, :]
            s = lax.dot_general(
                qh, kc, (((1,), (1,)), ((), ())),
                preferred_element_type=jnp.float32,
            ) * SCALE
            sm = jnp.where(live, s, NEG)
            m_old = m_ref[h]
            m_new = jnp.maximum(m_old, jnp.max(sm, axis=1, keepdims=True))
            alpha = jnp.exp(m_old - m_new)
            p = jnp.exp(sm - m_new) * wt
            l_ref[h] = l_ref[h] * alpha + jnp.sum(p, axis=1, keepdims=True)
            acc_ref[h] = acc_ref[h] * alpha + lax.dot_general(
                p, vc, (((1,), (0,)), ((), ())),
                preferred_element_type=jnp.float32,
            )
            m_ref[h] = m_new

        @pl.when(c == NC - 1)
        def _():
            partner = (1 - my_x, my_y, my_z)
            barrier = pltpu.get_barrier_semaphore()
            pl.semaphore_signal(barrier, inc=1, device_id=partner,
                                device_id_type=pl.DeviceIdType.MESH)
            pl.semaphore_wait(barrier, 1)

            copies = []
            for k_, (src, dst) in enumerate(
                    ((m_ref, rm_ref), (l_ref, rl_ref), (acc_ref, racc_ref))):
                rdma = pltpu.make_async_remote_copy(
                    src_ref=src, dst_ref=dst,
                    send_sem=send_sems.at[k_], recv_sem=recv_sems.at[k_],
                    device_id=partner, device_id_type=pl.DeviceIdType.MESH)
                rdma.start()
                copies.append(rdma)
            for rdma in copies:
                rdma.wait()

            m_l, l_l, a_l = m_ref[...], l_ref[...], acc_ref[...]
            m_r, l_r, a_r = rm_ref[...], rl_ref[...], racc_ref[...]
            m = jnp.maximum(m_l, m_r)
            c_l = jnp.exp(m_l - m)
            c_r = jnp.exp(m_r - m)
            l = l_l * c_l + l_r * c_r
            o = (a_l * c_l + a_r * c_r) / l
            out_ref[...] = jnp.transpose(o, (1, 0, 2)).reshape(B, 1, H, D)

    return pl.pallas_call(
        body,
        grid=(NC,),
        in_specs=[
            pl.BlockSpec((B, H, D), lambda c: (0, 0, 0)),
            pl.BlockSpec((CP, BS, H, D), lambda c: (c, 0, 0, 0)),
            pl.BlockSpec((CP, BS, H, D), lambda c: (c, 0, 0, 0)),
            pl.BlockSpec((B, NB), lambda c: (0, 0)),
            pl.BlockSpec((B, 1), lambda c: (0, 0)),
        ],
        out_specs=pl.BlockSpec((B, 1, H, D), lambda c: (0, 0, 0, 0)),
        out_shape=jax.ShapeDtypeStruct((B, 1, H, D), jnp.float32),
        scratch_shapes=[
            pltpu.VMEM((H, B, 1), jnp.float32),
            pltpu.VMEM((H, B, 1), jnp.float32),
            pltpu.VMEM((H, B, D), jnp.float32),
            pltpu.VMEM((H, B, 1), jnp.float32),
            pltpu.VMEM((H, B, 1), jnp.float32),
            pltpu.VMEM((H, B, D), jnp.float32),
            pltpu.SemaphoreType.DMA((3,)),
            pltpu.SemaphoreType.DMA((3,)),
        ],
        compiler_params=pltpu.CompilerParams(collective_id=0),
    )(q3, K, V, bt, lens2)


# device time: 150472 ns/iter; 1.0057x vs baseline; 1.0057x over previous
import jax
import jax.numpy as jnp
from jax import lax
from jax.experimental import pallas as pl
from jax.experimental.pallas import tpu as pltpu

B = 32
H = 16
D = 128
BS = 32
NB = 256
NPL = 256
CP = 16
T = CP * BS
NC = NPL // CP
SCALE = D ** -0.5
NEG = -1e30


def kernel(Q, K, V, bt, lens):
    q3 = Q.reshape(B, H, D)
    lens2 = lens.reshape(B, 1)

    def body(q_ref, k_ref, v_ref, bt_ref, lens_ref, out_ref,
             m_ref, l_ref, acc_ref, rm_ref, rl_ref, racc_ref,
             send_sems, recv_sems):
        c = pl.program_id(0)
        my_x = lax.axis_index("x")
        my_y = lax.axis_index("y")
        my_z = lax.axis_index("z")

        @pl.when(c == 0)
        def _():
            m_ref[...] = jnp.full((H, B, 1), NEG, jnp.float32)
            l_ref[...] = jnp.zeros((H, B, 1), jnp.float32)
            acc_ref[...] = jnp.zeros((H, B, D), jnp.float32)

        valid = lax.broadcasted_iota(jnp.int32, (B, NB), 1) < lens_ref[...]
        btv = jnp.where(valid, bt_ref[...], -1)
        base = my_x * NPL + c * CP
        pids = base + lax.broadcasted_iota(jnp.int32, (1, 1, CP), 2)
        w = jnp.sum((btv[:, :, None] == pids).astype(jnp.float32), axis=1)
        wt = jnp.broadcast_to(w[:, :, None], (B, CP, BS)).reshape(B, T)

        for h in range(H):
            kc = k_ref[:, :, h, :].reshape(T, D)
            vc = v_ref[:, :, h, :].reshape(T, D)
            qh = q_ref[:, h, :] * SCALE
            s = lax.dot_general(
                qh, kc, (((1,), (1,)), ((), ())),
                preferred_element_type=jnp.float32,
            )
            m_old = m_ref[h]
            m_new = jnp.maximum(m_old, jnp.max(s, axis=1, keepdims=True))
            alpha = jnp.exp(m_old - m_new)
            p = jnp.exp(s - m_new) * wt
            l_ref[h] = l_ref[h] * alpha + jnp.sum(p, axis=1, keepdims=True)
            acc_ref[h] = acc_ref[h] * alpha + lax.dot_general(
                p, vc, (((1,), (0,)), ((), ())),
                preferred_element_type=jnp.float32,
            )
            m_ref[h] = m_new

        @pl.when(c == NC - 1)
        def _():
            partner = (1 - my_x, my_y, my_z)
            barrier = pltpu.get_barrier_semaphore()
            pl.semaphore_signal(barrier, inc=1, device_id=partner,
                                device_id_type=pl.DeviceIdType.MESH)
            pl.semaphore_wait(barrier, 1)

            copies = []
            for k_, (src, dst) in enumerate(
                    ((m_ref, rm_ref), (l_ref, rl_ref), (acc_ref, racc_ref))):
                rdma = pltpu.make_async_remote_copy(
                    src_ref=src, dst_ref=dst,
                    send_sem=send_sems.at[k_], recv_sem=recv_sems.at[k_],
                    device_id=partner, device_id_type=pl.DeviceIdType.MESH)
                rdma.start()
                copies.append(rdma)
            for rdma in copies:
                rdma.wait()

            m_l, l_l, a_l = m_ref[...], l_ref[...], acc_ref[...]
            m_r, l_r, a_r = rm_ref[...], rl_ref[...], racc_ref[...]
            m = jnp.maximum(m_l, m_r)
            c_l = jnp.exp(m_l - m)
            c_r = jnp.exp(m_r - m)
            l = l_l * c_l + l_r * c_r
            o = (a_l * c_l + a_r * c_r) / l
            out_ref[...] = jnp.transpose(o, (1, 0, 2)).reshape(B, 1, H, D)

    return pl.pallas_call(
        body,
        grid=(NC,),
        in_specs=[
            pl.BlockSpec((B, H, D), lambda c: (0, 0, 0)),
            pl.BlockSpec((CP, BS, H, D), lambda c: (c, 0, 0, 0)),
            pl.BlockSpec((CP, BS, H, D), lambda c: (c, 0, 0, 0)),
            pl.BlockSpec((B, NB), lambda c: (0, 0)),
            pl.BlockSpec((B, 1), lambda c: (0, 0)),
        ],
        out_specs=pl.BlockSpec((B, 1, H, D), lambda c: (0, 0, 0, 0)),
        out_shape=jax.ShapeDtypeStruct((B, 1, H, D), jnp.float32),
        scratch_shapes=[
            pltpu.VMEM((H, B, 1), jnp.float32),
            pltpu.VMEM((H, B, 1), jnp.float32),
            pltpu.VMEM((H, B, D), jnp.float32),
            pltpu.VMEM((H, B, 1), jnp.float32),
            pltpu.VMEM((H, B, 1), jnp.float32),
            pltpu.VMEM((H, B, D), jnp.float32),
            pltpu.SemaphoreType.DMA((3,)),
            pltpu.SemaphoreType.DMA((3,)),
        ],
        compiler_params=pltpu.CompilerParams(collective_id=0),
    )(q3, K, V, bt, lens2)


# device time: 86975 ns/iter; 1.7400x vs baseline; 1.7301x over previous
import jax
import jax.numpy as jnp
from jax import lax
from jax.experimental import pallas as pl
from jax.experimental.pallas import tpu as pltpu

B = 32
H = 16
D = 128
BS = 32
NB = 256
NPL = 256
CP = 16
T = CP * BS
NR = 5
SCALE = D ** -0.5


def kernel(Q, K, V, bt, lens):
    q3 = Q.reshape(B, H, D)
    lens2 = lens.reshape(B, 1)

    def body(q_ref, k_hbm, v_hbm, bt_ref, lens_ref, out_ref,
             kbuf, vbuf, m_ref, l_ref, acc_ref, rm, rl, racc,
             load_sems, send_sems, recv_sems):
        my_x = lax.axis_index("x")
        my_y = lax.axis_index("y")
        my_z = lax.axis_index("z")
        slot = my_y * 4 + my_z

        kcopy = pltpu.make_async_copy(
            k_hbm.at[pl.ds(slot * CP, CP)], kbuf, load_sems.at[0])
        vcopy = pltpu.make_async_copy(
            v_hbm.at[pl.ds(slot * CP, CP)], vbuf, load_sems.at[1])
        kcopy.start()
        vcopy.start()

        partners = (
            (1 - my_x, my_y, my_z),
            (my_x, my_y ^ 1, my_z),
            (my_x, my_y ^ 2, my_z),
            (my_x, my_y, my_z ^ 1),
            (my_x, my_y, my_z ^ 2),
        )

        barrier = pltpu.get_barrier_semaphore()
        for p_ in partners:
            pl.semaphore_signal(barrier, inc=1, device_id=p_,
                                device_id_type=pl.DeviceIdType.MESH)
        pl.semaphore_wait(barrier, NR)

        valid = lax.broadcasted_iota(jnp.int32, (B, NB), 1) < lens_ref[...]
        btv = jnp.where(valid, bt_ref[...], -1)
        base = my_x * NPL + slot * CP
        pids = base + lax.broadcasted_iota(jnp.int32, (1, 1, CP), 2)
        w = jnp.sum((btv[:, :, None] == pids).astype(jnp.float32), axis=1)
        wt = jnp.broadcast_to(w[:, :, None], (B, CP, BS)).reshape(B, T)

        kcopy.wait()
        vcopy.wait()

        for h in range(H):
            kc = kbuf[:, :, h, :].reshape(T, D)
            vc = vbuf[:, :, h, :].reshape(T, D)
            qh = q_ref[:, h, :] * SCALE
            s = lax.dot_general(
                qh, kc, (((1,), (1,)), ((), ())),
                preferred_element_type=jnp.float32,
            )
            m = jnp.max(s, axis=1, keepdims=True)
            p = jnp.exp(s - m) * wt
            m_ref[h] = m
            l_ref[h] = jnp.sum(p, axis=1, keepdims=True)
            acc_ref[h] = lax.dot_general(
                p, vc, (((1,), (0,)), ((), ())),
                preferred_element_type=jnp.float32,
            )

        for r, partner in enumerate(partners):
            copies = []
            for j, (src, dst) in enumerate(
                    ((m_ref, rm.at[r]), (l_ref, rl.at[r]),
                     (acc_ref, racc.at[r]))):
                rdma = pltpu.make_async_remote_copy(
                    src_ref=src, dst_ref=dst,
                    send_sem=send_sems.at[r, j], recv_sem=recv_sems.at[r, j],
                    device_id=partner, device_id_type=pl.DeviceIdType.MESH)
                rdma.start()
                copies.append(rdma)
            for rdma in copies:
                rdma.wait()

            m_a, l_a, a_a = m_ref[...], l_ref[...], acc_ref[...]
            m_b, l_b, a_b = rm[r], rl[r], racc[r]
            m_n = jnp.maximum(m_a, m_b)
            c_a = jnp.exp(m_a - m_n)
            c_b = jnp.exp(m_b - m_n)
            m_ref[...] = m_n
            l_ref[...] = l_a * c_a + l_b * c_b
            acc_ref[...] = a_a * c_a + a_b * c_b

        o = acc_ref[...] / l_ref[...]
        out_ref[...] = jnp.transpose(o, (1, 0, 2)).reshape(B, 1, H, D)

    return pl.pallas_call(
        body,
        in_specs=[
            pl.BlockSpec(memory_space=pltpu.VMEM),
            pl.BlockSpec(memory_space=pl.ANY),
            pl.BlockSpec(memory_space=pl.ANY),
            pl.BlockSpec(memory_space=pltpu.VMEM),
            pl.BlockSpec(memory_space=pltpu.VMEM),
        ],
        out_specs=pl.BlockSpec(memory_space=pltpu.VMEM),
        out_shape=jax.ShapeDtypeStruct((B, 1, H, D), jnp.float32),
        scratch_shapes=[
            pltpu.VMEM((CP, BS, H, D), jnp.float32),
            pltpu.VMEM((CP, BS, H, D), jnp.float32),
            pltpu.VMEM((H, B, 1), jnp.float32),
            pltpu.VMEM((H, B, 1), jnp.float32),
            pltpu.VMEM((H, B, D), jnp.float32),
            pltpu.VMEM((NR, H, B, 1), jnp.float32),
            pltpu.VMEM((NR, H, B, 1), jnp.float32),
            pltpu.VMEM((NR, H, B, D), jnp.float32),
            pltpu.SemaphoreType.DMA((2,)),
            pltpu.SemaphoreType.DMA((NR, 3)),
            pltpu.SemaphoreType.DMA((NR, 3)),
        ],
        compiler_params=pltpu.CompilerParams(collective_id=0),
    )(q3, K, V, bt, lens2)


# device time: 35575 ns/iter; 4.2540x vs baseline; 2.4448x over previous
import jax
import jax.numpy as jnp
from jax import lax
from jax.experimental import pallas as pl
from jax.experimental.pallas import tpu as pltpu

B = 32
H = 16
HH = 8
D = 128
BS = 32
NB = 256
NPL = 256
CP = 16
T = CP * BS
NR = 5
SCALE = D ** -0.5


def kernel(Q, K, V, bt, lens):
    q3 = Q.reshape(B, H, D)
    lens2 = lens.reshape(B, 1)

    def body(q_ref, k_hbm, v_hbm, bt_ref, lens_ref, out_ref,
             kbuf, vbuf, m_ref, l_ref, acc_ref, abuf, rm, rl, racc,
             load_sems, send_sems, recv_sems):
        my_x = lax.axis_index("x")
        my_y = lax.axis_index("y")
        my_z = lax.axis_index("z")
        slot = my_y * 4 + my_z

        kcopy = pltpu.make_async_copy(
            k_hbm.at[pl.ds(slot * CP, CP)], kbuf, load_sems.at[0])
        vcopy = pltpu.make_async_copy(
            v_hbm.at[pl.ds(slot * CP, CP)], vbuf, load_sems.at[1])
        kcopy.start()
        vcopy.start()

        partners = (
            (1 - my_x, my_y, my_z),
            (my_x, my_y ^ 1, my_z),
            (my_x, my_y ^ 2, my_z),
            (my_x, my_y, my_z ^ 1),
            (my_x, my_y, my_z ^ 2),
        )

        barrier = pltpu.get_barrier_semaphore()
        for p_ in partners:
            pl.semaphore_signal(barrier, inc=1, device_id=p_,
                                device_id_type=pl.DeviceIdType.MESH)
        pl.semaphore_wait(barrier, NR)

        valid = lax.broadcasted_iota(jnp.int32, (B, NB), 1) < lens_ref[...]
        btv = jnp.where(valid, bt_ref[...], -1)
        base = my_x * NPL + slot * CP
        pids = base + lax.broadcasted_iota(jnp.int32, (CP, 1, 1), 0)
        wp = jnp.sum((btv[None] == pids).astype(jnp.float32), axis=2)
        w = jnp.transpose(wp)
        wt = jnp.broadcast_to(w[:, :, None], (B, CP, BS)).reshape(B, T)

        kcopy.wait()
        vcopy.wait()

        def exchange(r, g):
            abuf[g] = acc_ref[g].astype(jnp.bfloat16)
            descs = []
            for j, (src, dst) in enumerate(
                    ((m_ref.at[g], rm.at[r, g]), (l_ref.at[g], rl.at[r, g]),
                     (abuf.at[g], racc.at[r, g]))):
                rdma = pltpu.make_async_remote_copy(
                    src_ref=src, dst_ref=dst,
                    send_sem=send_sems.at[r, g, j],
                    recv_sem=recv_sems.at[r, g, j],
                    device_id=partners[r],
                    device_id_type=pl.DeviceIdType.MESH)
                rdma.start()
                descs.append(rdma)
            return descs

        inflight = [None, None]
        for g in range(2):
            for hh in range(HH):
                h = g * HH + hh
                kc = kbuf[:, :, h, :].reshape(T, D)
                vc = vbuf[:, :, h, :].reshape(T, D)
                qh = q_ref[:, h, :] * SCALE
                s = lax.dot_general(
                    qh, kc, (((1,), (1,)), ((), ())),
                    preferred_element_type=jnp.float32,
                )
                m = jnp.max(s, axis=1, keepdims=True)
                p = jnp.exp(s - m) * wt
                m_ref[g, :, hh:hh + 1] = m
                l_ref[g, :, hh:hh + 1] = jnp.sum(p, axis=1, keepdims=True)
                acc_ref[g, hh] = lax.dot_general(
                    p, vc, (((1,), (0,)), ((), ())),
                    preferred_element_type=jnp.float32,
                )
            inflight[g] = exchange(0, g)

        for r in range(NR):
            for g in range(2):
                for d_ in inflight[g]:
                    d_.wait()
                m_a, l_a, a_a = m_ref[g], l_ref[g], acc_ref[g]
                m_b, l_b = rm[r, g], rl[r, g]
                a_b = racc[r, g].astype(jnp.float32)
                m_n = jnp.maximum(m_a, m_b)
                c_a = jnp.exp(m_a - m_n)
                c_b = jnp.exp(m_b - m_n)
                m_ref[g] = m_n
                l_ref[g] = l_a * c_a + l_b * c_b
                acc_ref[g] = (a_a * jnp.transpose(c_a)[:, :, None]
                              + a_b * jnp.transpose(c_b)[:, :, None])
                if r + 1 < NR:
                    inflight[g] = exchange(r + 1, g)

        l_full = jnp.concatenate([l_ref[0], l_ref[1]], axis=1)
        o = acc_ref[...].reshape(H, B, D) / jnp.transpose(l_full)[:, :, None]
        out_ref[...] = jnp.transpose(o, (1, 0, 2)).reshape(B, 1, H, D)

    return pl.pallas_call(
        body,
        in_specs=[
            pl.BlockSpec(memory_space=pltpu.VMEM),
            pl.BlockSpec(memory_space=pl.ANY),
            pl.BlockSpec(memory_space=pl.ANY),
            pl.BlockSpec(memory_space=pltpu.VMEM),
            pl.BlockSpec(memory_space=pltpu.VMEM),
        ],
        out_specs=pl.BlockSpec(memory_space=pltpu.VMEM),
        out_shape=jax.ShapeDtypeStruct((B, 1, H, D), jnp.float32),
        scratch_shapes=[
            pltpu.VMEM((CP, BS, H, D), jnp.float32),
            pltpu.VMEM((CP, BS, H, D), jnp.float32),
            pltpu.VMEM((2, B, HH), jnp.float32),
            pltpu.VMEM((2, B, HH), jnp.float32),
            pltpu.VMEM((2, HH, B, D), jnp.float32),
            pltpu.VMEM((2, HH, B, D), jnp.bfloat16),
            pltpu.VMEM((NR, 2, B, HH), jnp.float32),
            pltpu.VMEM((NR, 2, B, HH), jnp.float32),
            pltpu.VMEM((NR, 2, HH, B, D), jnp.bfloat16),
            pltpu.SemaphoreType.DMA((2,)),
            pltpu.SemaphoreType.DMA((NR, 2, 3)),
            pltpu.SemaphoreType.DMA((NR, 2, 3)),
        ],
        compiler_params=pltpu.CompilerParams(collective_id=0),
    )(q3, K, V, bt, lens2)
